# baseline (speedup 1.0000x reference)
import jax
import jax.numpy as jnp
from jax import lax
from jax.experimental import pallas as pl
from jax.experimental.pallas import tpu as pltpu
from jax.experimental.pallas import tpu_sc as plsc

NUM_EMBEDDINGS = 1024
EMBEDDING_DIM = 64
COMMITMENT_COST = 0.25

_T = 1024
_N_TOKENS = 32 * 1024
_GRID = _N_TOKENS // _T
_INV_N = 1.0 / (_N_TOKENS * EMBEDDING_DIM)

_NC = 2
_NS = 16
_NW = _NC * _NS
_BPW = _N_TOKENS // _NW
_CH = 128
_PIECES = _BPW // _CH
_DPAD = 128

_T2 = 4096


def _vq_tc_kernel(z_ref, e_ref, idx_ref, loss_ref, epad_ref):
    i = pl.program_id(0)
    z = z_ref[0]
    e = e_ref[...]
    zz = jnp.sum(z * z, axis=1, keepdims=True)
    e2 = jnp.sum(e * e, axis=1)
    mm = jax.lax.dot_general(
        z, e, dimension_numbers=(((1,), (1,)), ((), ())),
        preferred_element_type=jnp.float32)
    d = zz - 2.0 * mm + e2[None, :]
    dmin = jnp.min(d, axis=1, keepdims=True)
    iota = jax.lax.broadcasted_iota(jnp.int32, d.shape, 1)
    idx = jnp.min(jnp.where(d == dmin, iota, NUM_EMBEDDINGS), axis=1)
    idx_ref[...] = idx
    part = jnp.sum(dmin, keepdims=True)

    @pl.when(i == 0)
    def _():
        loss_ref[...] = part
        epad_ref[...] = jnp.concatenate(
            [e, jnp.zeros((NUM_EMBEDDINGS, _DPAD - EMBEDDING_DIM),
                          jnp.float32)], axis=1)

    @pl.when(i > 0)
    def _():
        loss_ref[...] += part

    @pl.when(i == _GRID - 1)
    def _():
        m = loss_ref[...] * _INV_N
        loss_ref[...] = m + COMMITMENT_COST * m


def _sc_gather(e_hbm, idx_hbm, qpad_hbm, idx_v, rows0, rows1, g0, g1, o0, o1):
    wid = lax.axis_index("s") * _NC + lax.axis_index("c")
    rows = (rows0, rows1)
    gsem = (g0, g1)
    osem = (o0, o1)
    pltpu.sync_copy(idx_hbm.at[pl.ds(wid * _BPW, _BPW)], idx_v)

    def start_gather(p):
        b = p % 2
        pltpu.async_copy(
            e_hbm.at[idx_v.at[pl.ds(p * _CH, _CH)]], rows[b], gsem[b])

    def start_out(p):
        b = p % 2
        pltpu.async_copy(
            rows[b], qpad_hbm.at[pl.ds(wid * _BPW + p * _CH, _CH)], osem[b])

    start_gather(0)
    for p in range(_PIECES):
        b = p % 2
        if p + 1 < _PIECES:
            if p + 1 >= 2:
                pltpu.make_async_copy(
                    rows[(p + 1) % 2],
                    qpad_hbm.at[pl.ds(wid * _BPW + (p - 1) * _CH, _CH)],
                    osem[(p + 1) % 2]).wait()
            start_gather(p + 1)
        pltpu.make_async_copy(
            e_hbm.at[idx_v.at[pl.ds(p * _CH, _CH)]], rows[b], gsem[b]).wait()
        start_out(p)
    for p in (_PIECES - 2, _PIECES - 1):
        b = p % 2
        pltpu.make_async_copy(
            rows[b], qpad_hbm.at[pl.ds(wid * _BPW + p * _CH, _CH)],
            osem[b]).wait()


_sc_call = pl.kernel(
    _sc_gather,
    out_type=jax.ShapeDtypeStruct((_N_TOKENS, _DPAD), jnp.float32),
    mesh=plsc.VectorSubcoreMesh(core_axis_name="c", subcore_axis_name="s"),
    scratch_types=[
        pltpu.VMEM((_BPW,), jnp.int32),
        pltpu.VMEM((_CH, _DPAD), jnp.float32),
        pltpu.VMEM((_CH, _DPAD), jnp.float32),
        pltpu.SemaphoreType.DMA,
        pltpu.SemaphoreType.DMA,
        pltpu.SemaphoreType.DMA,
        pltpu.SemaphoreType.DMA,
    ],
)


def _st_kernel(z_ref, qpad_ref, out_ref):
    z = z_ref[...]
    q = qpad_ref[...].reshape(z.shape[0], 1024, _DPAD)[:, :, :EMBEDDING_DIM]
    out_ref[...] = z + (q - z)


def kernel(inputs, embed_weight):
    idx, loss2, e_pad = pl.pallas_call(
        _vq_tc_kernel,
        grid=(_GRID,),
        in_specs=[
            pl.BlockSpec((1, _T, EMBEDDING_DIM), lambda i: (i, 0, 0)),
            pl.BlockSpec((NUM_EMBEDDINGS, EMBEDDING_DIM), lambda i: (0, 0)),
        ],
        out_specs=[
            pl.BlockSpec((_T,), lambda i: (i,)),
            pl.BlockSpec((1, 1), lambda i: (0, 0)),
            pl.BlockSpec((NUM_EMBEDDINGS, _DPAD), lambda i: (0, 0)),
        ],
        out_shape=[
            jax.ShapeDtypeStruct((_N_TOKENS,), jnp.int32),
            jax.ShapeDtypeStruct((1, 1), jnp.float32),
            jax.ShapeDtypeStruct((NUM_EMBEDDINGS, _DPAD), jnp.float32),
        ],
    )(inputs, embed_weight)
    q_pad = _sc_call(e_pad, idx)
    _B2 = _T2 // 1024
    qst = pl.pallas_call(
        _st_kernel,
        grid=(_N_TOKENS // _T2,),
        in_specs=[
            pl.BlockSpec((_B2, 1024, EMBEDDING_DIM), lambda i: (i, 0, 0)),
            pl.BlockSpec((_T2, _DPAD), lambda i: (i, 0)),
        ],
        out_specs=pl.BlockSpec((_B2, 1024, EMBEDDING_DIM), lambda i: (i, 0, 0)),
        out_shape=jax.ShapeDtypeStruct((32, 1024, EMBEDDING_DIM), jnp.float32),
    )(inputs, q_pad)
    return (qst, loss2[0, 0], idx)


# speedup vs baseline: 1.1507x; 1.1507x over previous
import jax
import jax.numpy as jnp
from jax import lax
from jax.experimental import pallas as pl
from jax.experimental.pallas import tpu as pltpu
from jax.experimental.pallas import tpu_sc as plsc

NUM_EMBEDDINGS = 1024
EMBEDDING_DIM = 64
COMMITMENT_COST = 0.25

_T = 1024
_N_TOKENS = 32 * 1024
_GRID = _N_TOKENS // _T
_INV_N = 1.0 / (_N_TOKENS * EMBEDDING_DIM)

_NC = 2
_NS = 16
_NW = _NC * _NS
_BPW = _N_TOKENS // _NW
_CH = 128
_PIECES = _BPW // _CH
_DPAD = 128

_T2 = 4096


def _vq_tc_kernel(z_ref, e_ref, idx_ref, loss_ref, epad_ref):
    i = pl.program_id(0)
    z = z_ref[...]
    e = e_ref[...]
    zz = jnp.sum(z * z, axis=1, keepdims=True)
    e2 = jnp.sum(e * e, axis=1)
    mm = jax.lax.dot_general(
        z, e, dimension_numbers=(((1,), (1,)), ((), ())),
        preferred_element_type=jnp.float32)
    d = zz - 2.0 * mm + e2[None, :]
    dmin = jnp.min(d, axis=1, keepdims=True)
    iota = jax.lax.broadcasted_iota(jnp.int32, d.shape, 1).astype(jnp.float32)
    idx_f = jnp.min(jnp.where(d == dmin, iota, float(NUM_EMBEDDINGS)),
                    axis=1)
    idx_ref[...] = idx_f.astype(jnp.int32)
    part = jnp.sum(dmin, keepdims=True)

    @pl.when(i == 0)
    def _():
        loss_ref[...] = part
        epad_ref[...] = jnp.concatenate(
            [e, jnp.zeros((NUM_EMBEDDINGS, _DPAD - EMBEDDING_DIM),
                          jnp.float32)], axis=1)

    @pl.when(i > 0)
    def _():
        loss_ref[...] += part

    @pl.when(i == _GRID - 1)
    def _():
        m = loss_ref[...] * _INV_N
        loss_ref[...] = m + COMMITMENT_COST * m


def _sc_gather(e_hbm, idx_hbm, qpad_hbm, idx_v, rows0, rows1, g0, g1, o0, o1):
    wid = lax.axis_index("s") * _NC + lax.axis_index("c")
    rows = (rows0, rows1)
    gsem = (g0, g1)
    osem = (o0, o1)
    pltpu.sync_copy(idx_hbm.at[pl.ds(wid * _BPW, _BPW)], idx_v)

    def start_gather(p):
        b = p % 2
        pltpu.async_copy(
            e_hbm.at[idx_v.at[pl.ds(p * _CH, _CH)]], rows[b], gsem[b])

    def start_out(p):
        b = p % 2
        pltpu.async_copy(
            rows[b], qpad_hbm.at[pl.ds(wid * _BPW + p * _CH, _CH)], osem[b])

    start_gather(0)
    for p in range(_PIECES):
        b = p % 2
        if p + 1 < _PIECES:
            if p + 1 >= 2:
                pltpu.make_async_copy(
                    rows[(p + 1) % 2],
                    qpad_hbm.at[pl.ds(wid * _BPW + (p - 1) * _CH, _CH)],
                    osem[(p + 1) % 2]).wait()
            start_gather(p + 1)
        pltpu.make_async_copy(
            e_hbm.at[idx_v.at[pl.ds(p * _CH, _CH)]], rows[b], gsem[b]).wait()
        start_out(p)
    for p in (_PIECES - 2, _PIECES - 1):
        b = p % 2
        pltpu.make_async_copy(
            rows[b], qpad_hbm.at[pl.ds(wid * _BPW + p * _CH, _CH)],
            osem[b]).wait()


_sc_call = pl.kernel(
    _sc_gather,
    out_type=jax.ShapeDtypeStruct((_N_TOKENS, _DPAD), jnp.float32),
    mesh=plsc.VectorSubcoreMesh(core_axis_name="c", subcore_axis_name="s"),
    scratch_types=[
        pltpu.VMEM((_BPW,), jnp.int32),
        pltpu.VMEM((_CH, _DPAD), jnp.float32),
        pltpu.VMEM((_CH, _DPAD), jnp.float32),
        pltpu.SemaphoreType.DMA,
        pltpu.SemaphoreType.DMA,
        pltpu.SemaphoreType.DMA,
        pltpu.SemaphoreType.DMA,
    ],
)


def _st_kernel(z_ref, qpad_ref, out_ref):
    z = z_ref[...]
    q = qpad_ref[:, :EMBEDDING_DIM]
    out_ref[...] = z + (q - z)


def kernel(inputs, embed_weight):
    flat = inputs.reshape(-1, EMBEDDING_DIM)
    idx, loss2, e_pad = pl.pallas_call(
        _vq_tc_kernel,
        grid=(_GRID,),
        in_specs=[
            pl.BlockSpec((_T, EMBEDDING_DIM), lambda i: (i, 0)),
            pl.BlockSpec((NUM_EMBEDDINGS, EMBEDDING_DIM), lambda i: (0, 0)),
        ],
        out_specs=[
            pl.BlockSpec((_T,), lambda i: (i,)),
            pl.BlockSpec((1, 1), lambda i: (0, 0)),
            pl.BlockSpec((NUM_EMBEDDINGS, _DPAD), lambda i: (0, 0)),
        ],
        out_shape=[
            jax.ShapeDtypeStruct((_N_TOKENS,), jnp.int32),
            jax.ShapeDtypeStruct((1, 1), jnp.float32),
            jax.ShapeDtypeStruct((NUM_EMBEDDINGS, _DPAD), jnp.float32),
        ],
    )(flat, embed_weight)
    q_pad = _sc_call(e_pad, idx)
    qst = pl.pallas_call(
        _st_kernel,
        grid=(_N_TOKENS // _T2,),
        in_specs=[
            pl.BlockSpec((_T2, EMBEDDING_DIM), lambda i: (i, 0)),
            pl.BlockSpec((_T2, _DPAD), lambda i: (i, 0)),
        ],
        out_specs=pl.BlockSpec((_T2, EMBEDDING_DIM), lambda i: (i, 0)),
        out_shape=jax.ShapeDtypeStruct((_N_TOKENS, EMBEDDING_DIM), jnp.float32),
    )(flat, q_pad)
    return (qst.reshape(inputs.shape), loss2[0, 0], idx)


# speedup vs baseline: 1.2023x; 1.0449x over previous
import jax
import jax.numpy as jnp
from jax import lax
from jax.experimental import pallas as pl
from jax.experimental.pallas import tpu as pltpu
from jax.experimental.pallas import tpu_sc as plsc

NUM_EMBEDDINGS = 1024
EMBEDDING_DIM = 64
COMMITMENT_COST = 0.25

_T = 1024
_N_TOKENS = 32 * 1024
_N_CHUNKS = 2
_C_TOKENS = _N_TOKENS // _N_CHUNKS
_C_GRID = _C_TOKENS // _T
_INV_N = 1.0 / (_N_TOKENS * EMBEDDING_DIM)

_NC = 2
_NS = 16
_NW = _NC * _NS
_BPW = _C_TOKENS // _NW
_CH = 128
_PIECES = _BPW // _CH
_DPAD = 128

_T2 = 4096


def _vq_tc_first(z_ref, e_ref, idx_ref, part_ref, epad_ref):
    i = pl.program_id(0)
    z = z_ref[...]
    e = e_ref[...]
    zz = jnp.sum(z * z, axis=1, keepdims=True)
    e2 = jnp.sum(e * e, axis=1)
    mm = jax.lax.dot_general(
        z, e, dimension_numbers=(((1,), (1,)), ((), ())),
        preferred_element_type=jnp.float32)
    d = zz - 2.0 * mm + e2[None, :]
    dmin = jnp.min(d, axis=1, keepdims=True)
    iota = jax.lax.broadcasted_iota(jnp.int32, d.shape, 1).astype(jnp.float32)
    idx_f = jnp.min(jnp.where(d == dmin, iota, float(NUM_EMBEDDINGS)), axis=1)
    idx_ref[...] = idx_f.astype(jnp.int32)
    part = jnp.sum(dmin, keepdims=True)

    @pl.when(i == 0)
    def _():
        part_ref[...] = part
        epad_ref[...] = jnp.concatenate(
            [e, jnp.zeros((NUM_EMBEDDINGS, _DPAD - EMBEDDING_DIM),
                          jnp.float32)], axis=1)

    @pl.when(i > 0)
    def _():
        part_ref[...] += part


def _vq_tc_last(z_ref, e_ref, prev_ref, idx_ref, loss_ref):
    i = pl.program_id(0)
    z = z_ref[...]
    e = e_ref[...]
    zz = jnp.sum(z * z, axis=1, keepdims=True)
    e2 = jnp.sum(e * e, axis=1)
    mm = jax.lax.dot_general(
        z, e, dimension_numbers=(((1,), (1,)), ((), ())),
        preferred_element_type=jnp.float32)
    d = zz - 2.0 * mm + e2[None, :]
    dmin = jnp.min(d, axis=1, keepdims=True)
    iota = jax.lax.broadcasted_iota(jnp.int32, d.shape, 1).astype(jnp.float32)
    idx_f = jnp.min(jnp.where(d == dmin, iota, float(NUM_EMBEDDINGS)), axis=1)
    idx_ref[...] = idx_f.astype(jnp.int32)
    part = jnp.sum(dmin, keepdims=True)

    @pl.when(i == 0)
    def _():
        loss_ref[...] = part

    @pl.when(i > 0)
    def _():
        loss_ref[...] += part

    @pl.when(i == _C_GRID - 1)
    def _():
        m = (loss_ref[...] + prev_ref[...]) * _INV_N
        loss_ref[...] = m + COMMITMENT_COST * m


def _sc_gather(e_hbm, idx_hbm, qpad_hbm, idx_v, rows0, rows1, g0, g1, o0, o1):
    wid = lax.axis_index("s") * _NC + lax.axis_index("c")
    rows = (rows0, rows1)
    gsem = (g0, g1)
    osem = (o0, o1)
    pltpu.sync_copy(idx_hbm.at[pl.ds(wid * _BPW, _BPW)], idx_v)

    def start_gather(p):
        b = p % 2
        pltpu.async_copy(
            e_hbm.at[idx_v.at[pl.ds(p * _CH, _CH)]], rows[b], gsem[b])

    def start_out(p):
        b = p % 2
        pltpu.async_copy(
            rows[b], qpad_hbm.at[pl.ds(wid * _BPW + p * _CH, _CH)], osem[b])

    start_gather(0)
    for p in range(_PIECES):
        b = p % 2
        if p + 1 < _PIECES:
            if p + 1 >= 2:
                pltpu.make_async_copy(
                    rows[(p + 1) % 2],
                    qpad_hbm.at[pl.ds(wid * _BPW + (p - 1) * _CH, _CH)],
                    osem[(p + 1) % 2]).wait()
            start_gather(p + 1)
        pltpu.make_async_copy(
            e_hbm.at[idx_v.at[pl.ds(p * _CH, _CH)]], rows[b], gsem[b]).wait()
        start_out(p)
    for p in (_PIECES - 2, _PIECES - 1):
        b = p % 2
        pltpu.make_async_copy(
            rows[b], qpad_hbm.at[pl.ds(wid * _BPW + p * _CH, _CH)],
            osem[b]).wait()


_sc_call = pl.kernel(
    _sc_gather,
    out_type=jax.ShapeDtypeStruct((_C_TOKENS, _DPAD), jnp.float32),
    mesh=plsc.VectorSubcoreMesh(core_axis_name="c", subcore_axis_name="s"),
    scratch_types=[
        pltpu.VMEM((_BPW,), jnp.int32),
        pltpu.VMEM((_CH, _DPAD), jnp.float32),
        pltpu.VMEM((_CH, _DPAD), jnp.float32),
        pltpu.SemaphoreType.DMA,
        pltpu.SemaphoreType.DMA,
        pltpu.SemaphoreType.DMA,
        pltpu.SemaphoreType.DMA,
    ],
)


def _st_first(z_ref, qpad_ref, out_ref):
    z = z_ref[...]
    q = qpad_ref[:, :EMBEDDING_DIM]
    out_ref[...] = z + (q - z)


def _st_last(z_ref, qpad_ref, prev_ref, out_ref):
    del prev_ref
    z = z_ref[...]
    q = qpad_ref[:, :EMBEDDING_DIM]
    out_ref[...] = z + (q - z)


def kernel(inputs, embed_weight):
    flat = inputs.reshape(-1, EMBEDDING_DIM)
    idx_a, part_a, e_pad = pl.pallas_call(
        _vq_tc_first,
        grid=(_C_GRID,),
        in_specs=[
            pl.BlockSpec((_T, EMBEDDING_DIM), lambda i: (i, 0)),
            pl.BlockSpec((NUM_EMBEDDINGS, EMBEDDING_DIM), lambda i: (0, 0)),
        ],
        out_specs=[
            pl.BlockSpec((_T,), lambda i: (i,)),
            pl.BlockSpec((1, 1), lambda i: (0, 0)),
            pl.BlockSpec((NUM_EMBEDDINGS, _DPAD), lambda i: (0, 0)),
        ],
        out_shape=[
            jax.ShapeDtypeStruct((_C_TOKENS,), jnp.int32),
            jax.ShapeDtypeStruct((1, 1), jnp.float32),
            jax.ShapeDtypeStruct((NUM_EMBEDDINGS, _DPAD), jnp.float32),
        ],
    )(flat, embed_weight)
    q_pad_a = _sc_call(e_pad, idx_a)
    idx_b, loss2 = pl.pallas_call(
        _vq_tc_last,
        grid=(_C_GRID,),
        in_specs=[
            pl.BlockSpec((_T, EMBEDDING_DIM), lambda i: (i + _C_GRID, 0)),
            pl.BlockSpec((NUM_EMBEDDINGS, EMBEDDING_DIM), lambda i: (0, 0)),
            pl.BlockSpec((1, 1), lambda i: (0, 0)),
        ],
        out_specs=[
            pl.BlockSpec((_T,), lambda i: (i,)),
            pl.BlockSpec((1, 1), lambda i: (0, 0)),
        ],
        out_shape=[
            jax.ShapeDtypeStruct((_C_TOKENS,), jnp.int32),
            jax.ShapeDtypeStruct((1, 1), jnp.float32),
        ],
    )(flat, embed_weight, part_a)
    q_pad_b = _sc_call(e_pad, idx_b)

    st_grid = _C_TOKENS // _T2
    qst_a = pl.pallas_call(
        _st_first,
        grid=(st_grid,),
        in_specs=[
            pl.BlockSpec((_T2, EMBEDDING_DIM), lambda i: (i, 0)),
            pl.BlockSpec((_T2, _DPAD), lambda i: (i, 0)),
        ],
        out_specs=pl.BlockSpec((_T2, EMBEDDING_DIM), lambda i: (i, 0)),
        out_shape=jax.ShapeDtypeStruct((_N_TOKENS, EMBEDDING_DIM), jnp.float32),
    )(flat, q_pad_a)
    qst = pl.pallas_call(
        _st_last,
        grid=(st_grid,),
        in_specs=[
            pl.BlockSpec((_T2, EMBEDDING_DIM), lambda i: (i + st_grid, 0)),
            pl.BlockSpec((_T2, _DPAD), lambda i: (i, 0)),
            pl.BlockSpec(memory_space=pl.ANY),
        ],
        out_specs=pl.BlockSpec((_T2, EMBEDDING_DIM), lambda i: (i + st_grid, 0)),
        out_shape=jax.ShapeDtypeStruct((_N_TOKENS, EMBEDDING_DIM), jnp.float32),
        input_output_aliases={2: 0},
    )(flat, q_pad_b, qst_a)
    idx = jnp.concatenate([idx_a, idx_b])
    return (qst.reshape(inputs.shape), loss2[0, 0], idx)


# speedup vs baseline: 1.2187x; 1.0136x over previous
import functools

import jax
import jax.numpy as jnp
from jax import lax
from jax.experimental import pallas as pl
from jax.experimental.pallas import tpu as pltpu
from jax.experimental.pallas import tpu_sc as plsc

NUM_EMBEDDINGS = 1024
EMBEDDING_DIM = 64
COMMITMENT_COST = 0.25

_T = 1024
_N_TOKENS = 32 * 1024
_N_CHUNKS = 4
_C_TOKENS = _N_TOKENS // _N_CHUNKS
_C_GRID = _C_TOKENS // _T
_INV_N = 1.0 / (_N_TOKENS * EMBEDDING_DIM)

_NC = 2
_NS = 16
_NW = _NC * _NS
_BPW = _C_TOKENS // _NW
_CH = 128
_PIECES = _BPW // _CH
_DPAD = 128

_T2 = 4096


def _vq_tc_body(is_first, is_last, z_ref, e_ref, *refs):
    if is_first:
        idx_ref, part_ref, epad_ref = refs
        prev_ref = None
    else:
        prev_ref, idx_ref, part_ref = refs
    i = pl.program_id(0)
    z = z_ref[...]
    e = e_ref[...]
    zz = jnp.sum(z * z, axis=1, keepdims=True)
    e2 = jnp.sum(e * e, axis=1)
    mm = jax.lax.dot_general(
        z, e, dimension_numbers=(((1,), (1,)), ((), ())),
        preferred_element_type=jnp.float32)
    d = zz - 2.0 * mm + e2[None, :]
    dmin = jnp.min(d, axis=1, keepdims=True)
    iota = jax.lax.broadcasted_iota(jnp.int32, d.shape, 1).astype(jnp.float32)
    idx_f = jnp.min(jnp.where(d == dmin, iota, float(NUM_EMBEDDINGS)), axis=1)
    idx_ref[...] = idx_f.astype(jnp.int32)
    part = jnp.sum(dmin, keepdims=True)

    @pl.when(i == 0)
    def _():
        if is_first:
            part_ref[...] = part
            epad_ref[...] = jnp.concatenate(
                [e, jnp.zeros((NUM_EMBEDDINGS, _DPAD - EMBEDDING_DIM),
                              jnp.float32)], axis=1)
        else:
            part_ref[...] = prev_ref[...] + part

    @pl.when(i > 0)
    def _():
        part_ref[...] += part

    if is_last:
        @pl.when(i == _C_GRID - 1)
        def _():
            m = part_ref[...] * _INV_N
            part_ref[...] = m + COMMITMENT_COST * m


def _sc_gather(e_hbm, idx_hbm, qpad_hbm, idx_v, rows0, rows1, g0, g1, o0, o1):
    wid = lax.axis_index("s") * _NC + lax.axis_index("c")
    rows = (rows0, rows1)
    gsem = (g0, g1)
    osem = (o0, o1)
    pltpu.sync_copy(idx_hbm.at[pl.ds(wid * _BPW, _BPW)], idx_v)

    def start_gather(p):
        b = p % 2
        pltpu.async_copy(
            e_hbm.at[idx_v.at[pl.ds(p * _CH, _CH)]], rows[b], gsem[b])

    def start_out(p):
        b = p % 2
        pltpu.async_copy(
            rows[b], qpad_hbm.at[pl.ds(wid * _BPW + p * _CH, _CH)], osem[b])

    start_gather(0)
    for p in range(_PIECES):
        b = p % 2
        if p + 1 < _PIECES:
            if p + 1 >= 2:
                pltpu.make_async_copy(
                    rows[(p + 1) % 2],
                    qpad_hbm.at[pl.ds(wid * _BPW + (p - 1) * _CH, _CH)],
                    osem[(p + 1) % 2]).wait()
            start_gather(p + 1)
        pltpu.make_async_copy(
            e_hbm.at[idx_v.at[pl.ds(p * _CH, _CH)]], rows[b], gsem[b]).wait()
        start_out(p)
    for p in range(max(_PIECES - 2, 0), _PIECES):
        b = p % 2
        pltpu.make_async_copy(
            rows[b], qpad_hbm.at[pl.ds(wid * _BPW + p * _CH, _CH)],
            osem[b]).wait()


_sc_call = pl.kernel(
    _sc_gather,
    out_type=jax.ShapeDtypeStruct((_C_TOKENS, _DPAD), jnp.float32),
    mesh=plsc.VectorSubcoreMesh(core_axis_name="c", subcore_axis_name="s"),
    scratch_types=[
        pltpu.VMEM((_BPW,), jnp.int32),
        pltpu.VMEM((_CH, _DPAD), jnp.float32),
        pltpu.VMEM((_CH, _DPAD), jnp.float32),
        pltpu.SemaphoreType.DMA,
        pltpu.SemaphoreType.DMA,
        pltpu.SemaphoreType.DMA,
        pltpu.SemaphoreType.DMA,
    ],
)


def _st_first(z_ref, qpad_ref, out_ref):
    z = z_ref[...]
    q = qpad_ref[:, :EMBEDDING_DIM]
    out_ref[...] = z + (q - z)


def _st_next(z_ref, qpad_ref, prev_ref, out_ref):
    del prev_ref
    z = z_ref[...]
    q = qpad_ref[:, :EMBEDDING_DIM]
    out_ref[...] = z + (q - z)


def _tc_call(chunk):
    is_first = chunk == 0
    is_last = chunk == _N_CHUNKS - 1
    base = chunk * _C_GRID
    in_specs = [
        pl.BlockSpec((_T, EMBEDDING_DIM), lambda i: (i + base, 0)),
        pl.BlockSpec((NUM_EMBEDDINGS, EMBEDDING_DIM), lambda i: (0, 0)),
    ]
    out_specs = [
        pl.BlockSpec((_T,), lambda i: (i,)),
        pl.BlockSpec((1, 1), lambda i: (0, 0)),
    ]
    out_shape = [
        jax.ShapeDtypeStruct((_C_TOKENS,), jnp.int32),
        jax.ShapeDtypeStruct((1, 1), jnp.float32),
    ]
    if is_first:
        out_specs.append(pl.BlockSpec((NUM_EMBEDDINGS, _DPAD),
                                      lambda i: (0, 0)))
        out_shape.append(
            jax.ShapeDtypeStruct((NUM_EMBEDDINGS, _DPAD), jnp.float32))
    else:
        in_specs.append(pl.BlockSpec((1, 1), lambda i: (0, 0)))
    return pl.pallas_call(
        functools.partial(_vq_tc_body, is_first, is_last),
        grid=(_C_GRID,),
        in_specs=in_specs,
        out_specs=out_specs,
        out_shape=out_shape,
    )


def kernel(inputs, embed_weight):
    flat = inputs.reshape(-1, EMBEDDING_DIM)
    idx_chunks = []
    qpad_chunks = []
    part = None
    e_pad = None
    for c in range(_N_CHUNKS):
        call = _tc_call(c)
        if c == 0:
            idx_c, part, e_pad = call(flat, embed_weight)
        else:
            idx_c, part = call(flat, embed_weight, part)
        idx_chunks.append(idx_c)
        qpad_chunks.append(_sc_call(e_pad, idx_c))
    loss2 = part

    st_grid = _C_TOKENS // _T2
    qst = None
    for c in range(_N_CHUNKS):
        off = c * st_grid
        if c == 0:
            qst = pl.pallas_call(
                _st_first,
                grid=(st_grid,),
                in_specs=[
                    pl.BlockSpec((_T2, EMBEDDING_DIM), lambda i: (i, 0)),
                    pl.BlockSpec((_T2, _DPAD), lambda i: (i, 0)),
                ],
                out_specs=pl.BlockSpec((_T2, EMBEDDING_DIM),
                                       lambda i: (i, 0)),
                out_shape=jax.ShapeDtypeStruct(
                    (_N_TOKENS, EMBEDDING_DIM), jnp.float32),
            )(flat, qpad_chunks[0])
        else:
            qst = pl.pallas_call(
                _st_next,
                grid=(st_grid,),
                in_specs=[
                    pl.BlockSpec((_T2, EMBEDDING_DIM),
                                 lambda i, off=off: (i + off, 0)),
                    pl.BlockSpec((_T2, _DPAD), lambda i: (i, 0)),
                    pl.BlockSpec(memory_space=pl.ANY),
                ],
                out_specs=pl.BlockSpec((_T2, EMBEDDING_DIM),
                                       lambda i, off=off: (i + off, 0)),
                out_shape=jax.ShapeDtypeStruct(
                    (_N_TOKENS, EMBEDDING_DIM), jnp.float32),
                input_output_aliases={2: 0},
            )(flat, qpad_chunks[c], qst)
    idx = jnp.concatenate(idx_chunks)
    return (qst.reshape(inputs.shape), loss2[0, 0], idx)


# speedup vs baseline: 1.2828x; 1.0526x over previous
import functools

import jax
import jax.numpy as jnp
from jax import lax
from jax.experimental import pallas as pl
from jax.experimental.pallas import tpu as pltpu
from jax.experimental.pallas import tpu_sc as plsc

NUM_EMBEDDINGS = 1024
EMBEDDING_DIM = 64
COMMITMENT_COST = 0.25

_T = 1024
_N_TOKENS = 32 * 1024
_N_CHUNKS = 4
_C_TOKENS = _N_TOKENS // _N_CHUNKS
_C_GRID = _C_TOKENS // _T
_INV_N = 1.0 / (_N_TOKENS * EMBEDDING_DIM)

_NC = 2
_NS = 16
_NW = _NC * _NS
_BPW = _C_TOKENS // _NW
_CH = 128
_PIECES = _BPW // _CH
_DPAD = 128
_LANES = 16


def _vq_tc_body(is_first, is_last, z_ref, e_ref, *refs):
    if is_first:
        idx_ref, part_ref, epad_ref = refs
        prev_ref = None
    else:
        prev_ref, idx_ref, part_ref = refs
    i = pl.program_id(0)
    z = z_ref[...]
    e = e_ref[...]
    zz = jnp.sum(z * z, axis=1, keepdims=True)
    e2 = jnp.sum(e * e, axis=1)
    mm = jax.lax.dot_general(
        z, e, dimension_numbers=(((1,), (1,)), ((), ())),
        preferred_element_type=jnp.float32)
    d = zz - 2.0 * mm + e2[None, :]
    dmin = jnp.min(d, axis=1, keepdims=True)
    iota = jax.lax.broadcasted_iota(jnp.int32, d.shape, 1).astype(jnp.float32)
    idx_f = jnp.min(jnp.where(d == dmin, iota, float(NUM_EMBEDDINGS)), axis=1)
    idx_ref[...] = idx_f.astype(jnp.int32)
    part = jnp.sum(dmin, keepdims=True)

    @pl.when(i == 0)
    def _():
        if is_first:
            part_ref[...] = part
            epad_ref[...] = jnp.concatenate(
                [e, jnp.zeros((NUM_EMBEDDINGS, _DPAD - EMBEDDING_DIM),
                              jnp.float32)], axis=1)
        else:
            part_ref[...] = prev_ref[...] + part

    @pl.when(i > 0)
    def _():
        part_ref[...] += part

    if is_last:
        @pl.when(i == _C_GRID - 1)
        def _():
            m = part_ref[...] * _INV_N
            part_ref[...] = m + COMMITMENT_COST * m


def _sc_gather_st(chunk_base, e_hbm, idx_hbm, z_hbm, qst_hbm,
                  idx_v, rows0, rows1, zb0, zb1,
                  g0, g1, s0, s1, o0, o1):
    wid = lax.axis_index("s") * _NC + lax.axis_index("c")
    rows = (rows0, rows1)
    zbs = (zb0, zb1)
    gsem = (g0, g1)
    zsem = (s0, s1)
    osem = (o0, o1)
    base = chunk_base + wid * _BPW
    pltpu.sync_copy(idx_hbm.at[pl.ds(wid * _BPW, _BPW)], idx_v)

    for p in range(_PIECES):
        b = p % 2
        pltpu.async_copy(
            e_hbm.at[idx_v.at[pl.ds(p * _CH, _CH)]], rows[b], gsem[b])
        pltpu.async_copy(
            z_hbm.at[pl.ds(base + p * _CH, _CH)], zbs[b], zsem[b])
    for p in range(_PIECES):
        b = p % 2
        pltpu.make_async_copy(
            e_hbm.at[idx_v.at[pl.ds(p * _CH, _CH)]], rows[b], gsem[b]).wait()
        pltpu.make_async_copy(
            z_hbm.at[pl.ds(base + p * _CH, _CH)], zbs[b], zsem[b]).wait()
        rv = rows[b]
        zv = zbs[b]

        def body(t, carry):
            for u in range(2):
                for j in range(EMBEDDING_DIM // _LANES):
                    sl = pl.ds(j * _LANES, _LANES)
                    q = rv[2 * t + u, sl]
                    zz16 = zv[2 * t + u, sl]
                    zv[2 * t + u, sl] = zz16 + (q - zz16)
            return carry

        lax.fori_loop(0, _CH // 2, body, 0)
        pltpu.async_copy(zbs[b], qst_hbm.at[pl.ds(base + p * _CH, _CH)],
                         osem[b])
    for p in range(_PIECES):
        b = p % 2
        pltpu.make_async_copy(
            zbs[b], qst_hbm.at[pl.ds(base + p * _CH, _CH)], osem[b]).wait()


def _make_sc_call(chunk):
    return pl.kernel(
        functools.partial(_sc_gather_st, chunk * _C_TOKENS),
        out_type=(),
        mesh=plsc.VectorSubcoreMesh(core_axis_name="c", subcore_axis_name="s"),
        scratch_types=[
            pltpu.VMEM((_BPW,), jnp.int32),
            pltpu.VMEM((_CH, _DPAD), jnp.float32),
            pltpu.VMEM((_CH, _DPAD), jnp.float32),
            pltpu.VMEM((_CH, EMBEDDING_DIM), jnp.float32),
            pltpu.VMEM((_CH, EMBEDDING_DIM), jnp.float32),
            pltpu.SemaphoreType.DMA,
            pltpu.SemaphoreType.DMA,
            pltpu.SemaphoreType.DMA,
            pltpu.SemaphoreType.DMA,
            pltpu.SemaphoreType.DMA,
            pltpu.SemaphoreType.DMA,
        ],
    )


_sc_calls = [_make_sc_call(c) for c in range(_N_CHUNKS)]


def _tc_call(chunk):
    is_first = chunk == 0
    is_last = chunk == _N_CHUNKS - 1
    base = chunk * _C_GRID
    in_specs = [
        pl.BlockSpec((_T, EMBEDDING_DIM), lambda i: (i + base, 0)),
        pl.BlockSpec((NUM_EMBEDDINGS, EMBEDDING_DIM), lambda i: (0, 0)),
    ]
    out_specs = [
        pl.BlockSpec((_T,), lambda i: (i,)),
        pl.BlockSpec((1, 1), lambda i: (0, 0)),
    ]
    out_shape = [
        jax.ShapeDtypeStruct((_C_TOKENS,), jnp.int32),
        jax.ShapeDtypeStruct((1, 1), jnp.float32),
    ]
    if is_first:
        out_specs.append(pl.BlockSpec((NUM_EMBEDDINGS, _DPAD),
                                      lambda i: (0, 0)))
        out_shape.append(
            jax.ShapeDtypeStruct((NUM_EMBEDDINGS, _DPAD), jnp.float32))
    else:
        in_specs.append(pl.BlockSpec((1, 1), lambda i: (0, 0)))
    return pl.pallas_call(
        functools.partial(_vq_tc_body, is_first, is_last),
        grid=(_C_GRID,),
        in_specs=in_specs,
        out_specs=out_specs,
        out_shape=out_shape,
    )


def kernel(inputs, embed_weight):
    flat = inputs.reshape(-1, EMBEDDING_DIM)
    qst_ref = jax.new_ref(
        jnp.zeros((_N_TOKENS, EMBEDDING_DIM), jnp.float32))
    idx_chunks = []
    part = None
    e_pad = None
    for c in range(_N_CHUNKS):
        call = _tc_call(c)
        if c == 0:
            idx_c, part, e_pad = call(flat, embed_weight)
        else:
            idx_c, part = call(flat, embed_weight, part)
        idx_chunks.append(idx_c)
        _sc_calls[c](e_pad, idx_c, flat, qst_ref)
    idx = jnp.concatenate(idx_chunks)
    return (qst_ref[...].reshape(inputs.shape), part[0, 0], idx)


# speedup vs baseline: 1.3350x; 1.0407x over previous
import functools

import jax
import jax.numpy as jnp
from jax import lax
from jax.experimental import pallas as pl
from jax.experimental.pallas import tpu as pltpu
from jax.experimental.pallas import tpu_sc as plsc

NUM_EMBEDDINGS = 1024
EMBEDDING_DIM = 64
COMMITMENT_COST = 0.25

_T = 1024
_N_TOKENS = 32 * 1024
_N_CHUNKS = 4
_C_TOKENS = _N_TOKENS // _N_CHUNKS
_C_GRID = _C_TOKENS // _T
_INV_N = 1.0 / (_N_TOKENS * EMBEDDING_DIM)

_NC = 2
_NS = 16
_NW = _NC * _NS
_BPW = _C_TOKENS // _NW
_CH = 128
_PIECES = _BPW // _CH
_DPAD = 128
_LANES = 16


def _vq_tc_body(is_first, is_last, z_ref, e_ref, *refs):
    if is_first:
        idx_ref, part_ref, epad_ref = refs
        prev_ref = None
    else:
        prev_ref, idx_ref, part_ref = refs
    i = pl.program_id(0)
    z = z_ref[...]
    e = e_ref[...]
    zz = jnp.sum(z * z, axis=1, keepdims=True)
    e2 = jnp.sum(e * e, axis=1)
    mm = jax.lax.dot_general(
        z, e, dimension_numbers=(((1,), (1,)), ((), ())),
        preferred_element_type=jnp.float32)
    d = zz - 2.0 * mm + e2[None, :]
    dmin = jnp.min(d, axis=1, keepdims=True)
    iota = jax.lax.broadcasted_iota(jnp.int32, d.shape, 1).astype(jnp.float32)
    idx_f = jnp.min(jnp.where(d == dmin, iota, float(NUM_EMBEDDINGS)), axis=1)
    idx_ref[...] = idx_f.astype(jnp.int32)
    part = jnp.sum(dmin, keepdims=True)

    @pl.when(i == 0)
    def _():
        if is_first:
            part_ref[...] = part
            epad_ref[...] = jnp.concatenate(
                [e, jnp.zeros((NUM_EMBEDDINGS, _DPAD - EMBEDDING_DIM),
                              jnp.float32)], axis=1)
        else:
            part_ref[...] = prev_ref[...] + part

    @pl.when(i > 0)
    def _():
        part_ref[...] += part

    if is_last:
        @pl.when(i == _C_GRID - 1)
        def _():
            m = part_ref[...] * _INV_N
            part_ref[...] = m + COMMITMENT_COST * m


def _sc_gather_st(chunk_base, e_hbm, idx_hbm, z_hbm, qst_hbm,
                  idx_v, rows0, rows1, zb0, zb1,
                  g0, g1, s0, s1, o0, o1):
    wid = lax.axis_index("s") * _NC + lax.axis_index("c")
    rows = (rows0, rows1)
    zbs = (zb0, zb1)
    gsem = (g0, g1)
    zsem = (s0, s1)
    osem = (o0, o1)
    base = chunk_base + wid * _BPW
    pltpu.sync_copy(idx_hbm.at[pl.ds(wid * _BPW, _BPW)], idx_v)

    for p in range(_PIECES):
        b = p % 2
        pltpu.async_copy(
            e_hbm.at[idx_v.at[pl.ds(p * _CH, _CH)]], rows[b], gsem[b])
        pltpu.async_copy(
            z_hbm.at[pl.ds(base + p * _CH, _CH)], zbs[b], zsem[b])
    for p in range(_PIECES):
        b = p % 2
        pltpu.make_async_copy(
            e_hbm.at[idx_v.at[pl.ds(p * _CH, _CH)]], rows[b], gsem[b]).wait()
        pltpu.make_async_copy(
            z_hbm.at[pl.ds(base + p * _CH, _CH)], zbs[b], zsem[b]).wait()
        rv = rows[b]
        zv = zbs[b]

        def body(t, carry):
            for u in range(2):
                for j in range(EMBEDDING_DIM // _LANES):
                    sl = pl.ds(j * _LANES, _LANES)
                    q = rv[2 * t + u, sl]
                    zz16 = zv[2 * t + u, sl]
                    zv[2 * t + u, sl] = zz16 + (q - zz16)
            return carry

        lax.fori_loop(0, _CH // 2, body, 0)
        pltpu.async_copy(zbs[b], qst_hbm.at[pl.ds(base + p * _CH, _CH)],
                         osem[b])
    for p in range(_PIECES):
        b = p % 2
        pltpu.make_async_copy(
            zbs[b], qst_hbm.at[pl.ds(base + p * _CH, _CH)], osem[b]).wait()


def _make_sc_call(chunk):
    out_type = () if chunk else jax.ShapeDtypeStruct(
        (_N_TOKENS, EMBEDDING_DIM), jnp.float32)
    return pl.kernel(
        functools.partial(_sc_gather_st, chunk * _C_TOKENS),
        out_type=out_type,
        mesh=plsc.VectorSubcoreMesh(core_axis_name="c", subcore_axis_name="s"),
        scratch_types=[
            pltpu.VMEM((_BPW,), jnp.int32),
            pltpu.VMEM((_CH, _DPAD), jnp.float32),
            pltpu.VMEM((_CH, _DPAD), jnp.float32),
            pltpu.VMEM((_CH, EMBEDDING_DIM), jnp.float32),
            pltpu.VMEM((_CH, EMBEDDING_DIM), jnp.float32),
            pltpu.SemaphoreType.DMA,
            pltpu.SemaphoreType.DMA,
            pltpu.SemaphoreType.DMA,
            pltpu.SemaphoreType.DMA,
            pltpu.SemaphoreType.DMA,
            pltpu.SemaphoreType.DMA,
        ],
    )


_sc_calls = [_make_sc_call(c) for c in range(_N_CHUNKS)]


def _tc_call(chunk):
    is_first = chunk == 0
    is_last = chunk == _N_CHUNKS - 1
    base = chunk * _C_GRID
    in_specs = [
        pl.BlockSpec((_T, EMBEDDING_DIM), lambda i: (i + base, 0)),
        pl.BlockSpec((NUM_EMBEDDINGS, EMBEDDING_DIM), lambda i: (0, 0)),
    ]
    out_specs = [
        pl.BlockSpec((_T,), lambda i: (i,)),
        pl.BlockSpec((1, 1), lambda i: (0, 0)),
    ]
    out_shape = [
        jax.ShapeDtypeStruct((_C_TOKENS,), jnp.int32),
        jax.ShapeDtypeStruct((1, 1), jnp.float32),
    ]
    if is_first:
        out_specs.append(pl.BlockSpec((NUM_EMBEDDINGS, _DPAD),
                                      lambda i: (0, 0)))
        out_shape.append(
            jax.ShapeDtypeStruct((NUM_EMBEDDINGS, _DPAD), jnp.float32))
    else:
        in_specs.append(pl.BlockSpec((1, 1), lambda i: (0, 0)))
    return pl.pallas_call(
        functools.partial(_vq_tc_body, is_first, is_last),
        grid=(_C_GRID,),
        in_specs=in_specs,
        out_specs=out_specs,
        out_shape=out_shape,
    )


def kernel(inputs, embed_weight):
    flat = inputs.reshape(-1, EMBEDDING_DIM)
    qst_ref = None
    idx_chunks = []
    part = None
    e_pad = None
    for c in range(_N_CHUNKS):
        call = _tc_call(c)
        if c == 0:
            idx_c, part, e_pad = call(flat, embed_weight)
        else:
            idx_c, part = call(flat, embed_weight, part)
        idx_chunks.append(idx_c)
        if c == 0:
            qst0 = _sc_calls[0](e_pad, idx_c, flat)
            qst_ref = jax.new_ref(qst0)
        else:
            _sc_calls[c](e_pad, idx_c, flat, qst_ref)
    idx = jnp.concatenate(idx_chunks)
    return (qst_ref[...].reshape(inputs.shape), part[0, 0], idx)
